# Initial kernel scaffold; baseline (speedup 1.0000x reference)
import jax
import jax.numpy as jnp
from jax.experimental import pallas as pl


def kernel(node_repr, edge_repr, edge_src, edge_dst, rel_ids, edge_relative_time, target_local_idx, node_subgraph_id, edge_subgraph_id, node_hop_depth, W_scalar, b_scalar, W1, b1, W2, b2):
    raise NotImplementedError("write your pallas kernel here")


# speedup vs baseline: 80.8735x; 80.8735x over previous
import jax
import jax.numpy as jnp
from jax import lax
from jax.experimental import pallas as pl
from jax.experimental.pallas import tpu as pltpu
from jax.experimental.pallas import tpu_sc as plsc

HIDDEN = 128
N_NODES = 10000
N_EDGES = 320000
N_TARGETS = 1024
NQ = 16
EPW = N_EDGES // 32
NODE_CHUNK = 320


def _sc_kernel(egid_hbm, edst_hbm, erel_hbm, et_hbm, esrc_hbm, tli_hbm,
               ngid_hbm, ndepth_hbm, erepr_hbm, nrepr_hbm,
               pooled_out, scal_out,
               egid_v, edst_v, tli_v, ngid_v, ndepth_v,
               erows_v, nrows_v, relbuf_v, tbuf_v, srcbuf_v, gidbuf_v,
               ngidbuf_v, wbuf_v,
               eb0_v, eb1_v, eb2_v, eb3_v, nb0_v, nb1_v,
               accs_v, zbuf_v,
               acc_in_s, acc_out_s, acc_in_l, acc_out_l,
               acc_h1s, acc_h1l, acc_hop1, acc_hop2,
               sem):
    c = lax.axis_index("c")
    s = lax.axis_index("s")
    wid = c * 16 + s
    zero16 = jnp.zeros((16,), jnp.float32)

    accs = [acc_in_s, acc_out_s, acc_in_l, acc_out_l,
            acc_h1s, acc_h1l, acc_hop1, acc_hop2]

    def _zero_zbuf(i, _):
        for j in range(8):
            zbuf_v[i, pl.ds(j * 16, 16)] = zero16
        return 0

    lax.fori_loop(0, 64, _zero_zbuf, 0)

    def _zero_accs(i, _):
        for q in range(NQ):
            accs_v[q, pl.ds(i * 16, 16)] = zero16
        return 0

    lax.fori_loop(0, N_TARGETS // 16, _zero_accs, 0)

    for a in accs:
        pltpu.sync_copy(zbuf_v, a.at[pl.ds(s * 64, 64)])
    plsc.subcore_barrier()

    e0 = wid * EPW
    pltpu.sync_copy(egid_hbm.at[pl.ds(e0, EPW)], egid_v)
    pltpu.sync_copy(edst_hbm.at[pl.ds(e0, EPW)], edst_v)
    pltpu.sync_copy(tli_hbm, tli_v)
    n0 = wid * NODE_CHUNK

    @pl.when(wid < 31)
    def _():
        pltpu.sync_copy(ngid_hbm.at[pl.ds(n0, NODE_CHUNK)], ngid_v)
        pltpu.sync_copy(ndepth_hbm.at[pl.ds(n0, NODE_CHUNK)], ndepth_v)

    @pl.when(wid == 31)
    def _():
        pltpu.sync_copy(ngid_hbm.at[pl.ds(n0, 80)], ngid_v.at[pl.ds(0, 80)])
        pltpu.sync_copy(ndepth_hbm.at[pl.ds(n0, 80)],
                        ndepth_v.at[pl.ds(0, 80)])

    def edge_body(k, _):
        gid = egid_v[pl.ds(k * 16, 16)]
        dst = edst_v[pl.ds(k * 16, 16)]
        tgt = plsc.load_gather(tli_v, [gid])
        tem = dst == tgt
        nmatch = jnp.sum(tem.astype(jnp.int32))

        @pl.when(nmatch > 0)
        def _():
            ebase = e0 + k * 16
            pltpu.sync_copy(erel_hbm.at[pl.ds(ebase, 16)], relbuf_v)
            pltpu.sync_copy(et_hbm.at[pl.ds(ebase, 16)], tbuf_v)
            pltpu.sync_copy(esrc_hbm.at[pl.ds(ebase, 16)], srcbuf_v)
            rel = relbuf_v[...]
            t = tbuf_v[...]
            temf = tem.astype(jnp.float32)
            sw = jnp.exp(-t)
            lw = jnp.exp(t * (-0.1))
            inbf = jnp.where(rel < 4, temf, 0.0)
            outbf = temf - inbf
            w_in_s = sw * inbf
            w_out_s = sw * outbf
            w_in_l = lw * inbf
            w_out_l = lw * outbf
            w_s = sw * temf
            w_l = lw * temf
            gidbuf_v[...] = gid
            lanes = lax.broadcasted_iota(jnp.int32, (16,), 0)
            qs = [temf, t * temf, inbf, outbf,
                  w_in_s, w_out_s, w_in_l, w_out_l]
            for q in range(8):
                qrow = jnp.full((16,), q, jnp.int32)
                for i in range(16):
                    plsc.addupdate_scatter(accs_v, [qrow, gid], qs[q],
                                           mask=lanes == i)
            pltpu.sync_copy(erepr_hbm.at[pl.ds(ebase, 16)], erows_v)
            pltpu.async_copy(nrepr_hbm.at[srcbuf_v], nrows_v, sem).wait()
            for i in range(16):
                for j in range(8):
                    er = erows_v[i, pl.ds(j * 16, 16)]
                    nr = nrows_v[i, pl.ds(j * 16, 16)]
                    eb0_v[i, pl.ds(j * 16, 16)] = w_in_s[i] * er
                    eb1_v[i, pl.ds(j * 16, 16)] = w_out_s[i] * er
                    eb2_v[i, pl.ds(j * 16, 16)] = w_in_l[i] * er
                    eb3_v[i, pl.ds(j * 16, 16)] = w_out_l[i] * er
                    nb0_v[i, pl.ds(j * 16, 16)] = w_s[i] * nr
                    nb1_v[i, pl.ds(j * 16, 16)] = w_l[i] * nr
            pltpu.sync_copy(eb0_v, acc_in_s.at[gidbuf_v], add=True)
            pltpu.sync_copy(eb1_v, acc_out_s.at[gidbuf_v], add=True)
            pltpu.sync_copy(eb2_v, acc_in_l.at[gidbuf_v], add=True)
            pltpu.sync_copy(eb3_v, acc_out_l.at[gidbuf_v], add=True)
            pltpu.sync_copy(nb0_v, acc_h1s.at[gidbuf_v], add=True)
            pltpu.sync_copy(nb1_v, acc_h1l.at[gidbuf_v], add=True)
        return 0

    lax.fori_loop(0, EPW // 16, edge_body, 0)

    nvreg = jnp.where(wid < 31, NODE_CHUNK // 16, 80 // 16)

    def node_body(k, _):
        gid = ngid_v[pl.ds(k * 16, 16)]
        depth = ndepth_v[pl.ds(k * 16, 16)]
        hop1 = (depth == 1).astype(jnp.float32)
        hop2 = (depth >= 2).astype(jnp.float32)
        ngidbuf_v[...] = gid
        pltpu.sync_copy(nrepr_hbm.at[pl.ds(n0 + k * 16, 16)], nrows_v)
        lanes = lax.broadcasted_iota(jnp.int32, (16,), 0)
        row8 = jnp.full((16,), 8, jnp.int32)
        row9 = jnp.full((16,), 9, jnp.int32)
        for i in range(16):
            plsc.addupdate_scatter(accs_v, [row8, gid], hop1,
                                   mask=lanes == i)
            plsc.addupdate_scatter(accs_v, [row9, gid], hop2,
                                   mask=lanes == i)
        for i in range(16):
            for j in range(8):
                nr = nrows_v[i, pl.ds(j * 16, 16)]
                nb0_v[i, pl.ds(j * 16, 16)] = hop1[i] * nr
                nb1_v[i, pl.ds(j * 16, 16)] = hop2[i] * nr
        pltpu.sync_copy(nb0_v, acc_hop1.at[ngidbuf_v], add=True)
        pltpu.sync_copy(nb1_v, acc_hop2.at[ngidbuf_v], add=True)
        return 0

    lax.fori_loop(0, nvreg, node_body, 0)

    pltpu.sync_copy(accs_v, scal_out.at[wid])
    plsc.subcore_barrier()
    for j, a in enumerate(accs):
        pltpu.sync_copy(a.at[pl.ds(s * 64, 64)],
                        pooled_out.at[c, j, pl.ds(s * 64, 64)])


def _sc_stage(node_repr, edge_repr, edge_src, edge_dst, rel_ids,
              edge_relative_time, target_local_idx, node_subgraph_id,
              node_hop_depth, edge_subgraph_id):
    mesh = plsc.VectorSubcoreMesh(core_axis_name="c", subcore_axis_name="s")
    f32 = jnp.float32
    i32 = jnp.int32
    fn = pl.kernel(
        _sc_kernel,
        mesh=mesh,
        compiler_params=pltpu.CompilerParams(needs_layout_passes=False),
        out_type=[
            jax.ShapeDtypeStruct((2, 8, N_TARGETS, HIDDEN), f32),
            jax.ShapeDtypeStruct((32, NQ, N_TARGETS), f32),
        ],
        scratch_types=[
            pltpu.VMEM((EPW,), i32),
            pltpu.VMEM((EPW,), i32),
            pltpu.VMEM((N_TARGETS,), i32),
            pltpu.VMEM((NODE_CHUNK,), i32),
            pltpu.VMEM((NODE_CHUNK,), i32),
            pltpu.VMEM((16, HIDDEN), f32),
            pltpu.VMEM((16, HIDDEN), f32),
            pltpu.VMEM((16,), i32),
            pltpu.VMEM((16,), f32),
            pltpu.VMEM((16,), i32),
            pltpu.VMEM((16,), i32),
            pltpu.VMEM((16,), i32),
            pltpu.VMEM((14, 16), f32),
            pltpu.VMEM((16, HIDDEN), f32),
            pltpu.VMEM((16, HIDDEN), f32),
            pltpu.VMEM((16, HIDDEN), f32),
            pltpu.VMEM((16, HIDDEN), f32),
            pltpu.VMEM((16, HIDDEN), f32),
            pltpu.VMEM((16, HIDDEN), f32),
            pltpu.VMEM((NQ, N_TARGETS), f32),
            pltpu.VMEM((64, HIDDEN), f32),
            pltpu.VMEM_SHARED((N_TARGETS, HIDDEN), f32),
            pltpu.VMEM_SHARED((N_TARGETS, HIDDEN), f32),
            pltpu.VMEM_SHARED((N_TARGETS, HIDDEN), f32),
            pltpu.VMEM_SHARED((N_TARGETS, HIDDEN), f32),
            pltpu.VMEM_SHARED((N_TARGETS, HIDDEN), f32),
            pltpu.VMEM_SHARED((N_TARGETS, HIDDEN), f32),
            pltpu.VMEM_SHARED((N_TARGETS, HIDDEN), f32),
            pltpu.VMEM_SHARED((N_TARGETS, HIDDEN), f32),
            pltpu.SemaphoreType.DMA,
        ],
    )
    return fn(edge_subgraph_id, edge_dst, rel_ids, edge_relative_time,
              edge_src, target_local_idx, node_subgraph_id,
              node_hop_depth, edge_repr, node_repr)


def _layer_norm(x, eps=1e-5):
    mu = jnp.mean(x, axis=-1, keepdims=True)
    var = jnp.mean((x - mu) ** 2, axis=-1, keepdims=True)
    return (x - mu) * lax.rsqrt(var + eps)


def _gelu(x):
    return x * 0.5 * (1.0 + lax.erf(x * 0.7071067811865476))


def _tc_kernel(pooled_ref, scal_ref, wsc_ref, bsc_ref, w1_ref, b1_ref,
               w2_ref, b2_ref, out_ref):
    pooled = pooled_ref[0] + pooled_ref[1]
    scal = jnp.sum(scal_ref[...], axis=0)
    eps = 1e-6
    sw_in, sw_out = scal[4], scal[5]
    lw_in, lw_out = scal[6], scal[7]
    smass = sw_in + sw_out
    lmass = lw_in + lw_out

    def mean(p, d):
        return p / jnp.maximum(d, eps)[:, None]

    in_s = mean(pooled[0], sw_in)
    out_s = mean(pooled[1], sw_out)
    in_l = mean(pooled[2], lw_in)
    out_l = mean(pooled[3], lw_out)
    h1e_s = mean(pooled[4], smass)
    h1e_l = mean(pooled[5], lmass)
    hop1m = mean(pooled[6], scal[8])
    hop2m = mean(pooled[7], scal[9])

    dgap = _layer_norm(out_l - in_l)
    feats = [
        _layer_norm(in_s + out_s - (in_l + out_l)),
        dgap,
        _layer_norm(hop1m - hop2m),
        _layer_norm(h1e_s - h1e_l),
        _layer_norm(jnp.abs(dgap)),
        _layer_norm(hop1m),
        _layer_norm(hop2m),
        _layer_norm(in_l + out_l),
    ]
    sf = jnp.stack([
        jnp.log1p(scal[2]), jnp.log1p(scal[3]),
        jnp.log1p(scal[8]), jnp.log1p(scal[9]),
        scal[1] / jnp.maximum(scal[0], eps),
        smass, lmass, smass - lmass,
    ], axis=-1)
    emb = _gelu(jnp.dot(sf, wsc_ref[...],
                        preferred_element_type=jnp.float32) + bsc_ref[...])
    h = jnp.dot(emb, w1_ref[pl.ds(8 * HIDDEN, 32)],
                preferred_element_type=jnp.float32)
    for k in range(8):
        h = h + jnp.dot(feats[k], w1_ref[pl.ds(k * HIDDEN, HIDDEN)],
                        preferred_element_type=jnp.float32)
    h = _gelu(h + b1_ref[...])
    out_ref[...] = 0.25 * jnp.tanh(
        jnp.dot(h, w2_ref[...], preferred_element_type=jnp.float32)
        + b2_ref[...])


def _tc_stage(pooled, scal, W_scalar, b_scalar, W1, b1, W2, b2):
    B = 256
    grid = N_TARGETS // B
    return pl.pallas_call(
        _tc_kernel,
        grid=(grid,),
        in_specs=[
            pl.BlockSpec((2, 8, B, HIDDEN), lambda i: (0, 0, i, 0)),
            pl.BlockSpec((32, NQ, B), lambda i: (0, 0, i)),
            pl.BlockSpec((8, 32), lambda i: (0, 0)),
            pl.BlockSpec((1, 32), lambda i: (0, 0)),
            pl.BlockSpec((8 * HIDDEN + 32, 2 * HIDDEN), lambda i: (0, 0)),
            pl.BlockSpec((1, 2 * HIDDEN), lambda i: (0, 0)),
            pl.BlockSpec((2 * HIDDEN, HIDDEN), lambda i: (0, 0)),
            pl.BlockSpec((1, HIDDEN), lambda i: (0, 0)),
        ],
        out_specs=pl.BlockSpec((B, HIDDEN), lambda i: (i, 0)),
        out_shape=jax.ShapeDtypeStruct((N_TARGETS, HIDDEN), jnp.float32),
    )(pooled, scal, W_scalar, b_scalar.reshape(1, -1), W1,
      b1.reshape(1, -1), W2, b2.reshape(1, -1))


def kernel(node_repr, edge_repr, edge_src, edge_dst, rel_ids,
           edge_relative_time, target_local_idx, node_subgraph_id,
           edge_subgraph_id, node_hop_depth, W_scalar, b_scalar,
           W1, b1, W2, b2):
    i32 = jnp.int32
    pooled, scal = _sc_stage(
        node_repr, edge_repr, edge_src.astype(i32), edge_dst.astype(i32),
        rel_ids.astype(i32), edge_relative_time,
        target_local_idx.astype(i32), node_subgraph_id.astype(i32),
        node_hop_depth.astype(i32), edge_subgraph_id.astype(i32))
    return _tc_stage(pooled, scal, W_scalar, b_scalar, W1, b1, W2, b2)


# speedup vs baseline: 99.5780x; 1.2313x over previous
import jax
import jax.numpy as jnp
from jax import lax
from jax.experimental import pallas as pl
from jax.experimental.pallas import tpu as pltpu
from jax.experimental.pallas import tpu_sc as plsc

HIDDEN = 128
N_NODES = 10000
N_EDGES = 320000
N_TARGETS = 1024
NQ = 10
EPW = N_EDGES // 32
NODE_CHUNK = 320


def _sc_kernel(egid_hbm, edst_hbm, erel_hbm, et_hbm, esrc_hbm, tli_hbm,
               ngid_hbm, ndepth_hbm, erepr_hbm, nrepr_hbm,
               pooled_out, scal_out,
               egid_v, edst_v, tli_v, ngid_v, ndepth_v,
               erows_v, nrows_v, relbuf_v, tbuf_v, srcbuf_v, gidbuf_v,
               gidbuf2_v, ngidbuf_v, ngidbuf64_v,
               eb0_v, eb1_v, nb0_v, nb1_v,
               accs_v, zbuf_v,
               acc_s, acc_l, acc_h1s, acc_h1l, acc_hop,
               sem):
    c = lax.axis_index("c")
    s = lax.axis_index("s")
    wid = c * 16 + s
    zero16 = jnp.zeros((16,), jnp.float32)

    def _zero_zbuf(i, _):
        for j in range(8):
            zbuf_v[i, pl.ds(j * 16, 16)] = zero16
        return 0

    lax.fori_loop(0, 64, _zero_zbuf, 0)

    def _zero_accs(i, _):
        for q in range(NQ):
            accs_v[q, pl.ds(i * 16, 16)] = zero16
        return 0

    lax.fori_loop(0, N_TARGETS // 16, _zero_accs, 0)

    zcps = [
        pltpu.async_copy(zbuf_v, acc_s.at[pl.ds(s * 128, 64)], sem),
        pltpu.async_copy(zbuf_v, acc_s.at[pl.ds(s * 128 + 64, 64)], sem),
        pltpu.async_copy(zbuf_v, acc_l.at[pl.ds(s * 128, 64)], sem),
        pltpu.async_copy(zbuf_v, acc_l.at[pl.ds(s * 128 + 64, 64)], sem),
        pltpu.async_copy(zbuf_v, acc_hop.at[pl.ds(s * 128, 64)], sem),
        pltpu.async_copy(zbuf_v, acc_hop.at[pl.ds(s * 128 + 64, 64)], sem),
        pltpu.async_copy(zbuf_v, acc_h1s.at[pl.ds(s * 64, 64)], sem),
        pltpu.async_copy(zbuf_v, acc_h1l.at[pl.ds(s * 64, 64)], sem),
    ]
    for cp in zcps:
        cp.wait()
    plsc.subcore_barrier()

    e0 = wid * EPW
    n0 = wid * NODE_CHUNK
    scps = [pltpu.async_copy(egid_hbm.at[pl.ds(e0, EPW)], egid_v, sem),
            pltpu.async_copy(edst_hbm.at[pl.ds(e0, EPW)], edst_v, sem),
            pltpu.async_copy(tli_hbm, tli_v, sem)]
    for cp in scps:
        cp.wait()

    @pl.when(wid < 31)
    def _():
        pltpu.sync_copy(ngid_hbm.at[pl.ds(n0, NODE_CHUNK)], ngid_v)
        pltpu.sync_copy(ndepth_hbm.at[pl.ds(n0, NODE_CHUNK)], ndepth_v)

    @pl.when(wid == 31)
    def _():
        pltpu.sync_copy(ngid_hbm.at[pl.ds(n0, 80)], ngid_v.at[pl.ds(0, 80)])
        pltpu.sync_copy(ndepth_hbm.at[pl.ds(n0, 80)],
                        ndepth_v.at[pl.ds(0, 80)])

    EB = 25

    def edge_slow(k, _):
        gid = egid_v[pl.ds(k * 16, 16)]
        dst = edst_v[pl.ds(k * 16, 16)]
        tgt = plsc.load_gather(tli_v, [gid])
        tem = dst == tgt

        @pl.when(jnp.any(tem))
        def _():
            ebase = e0 + k * 16
            pltpu.sync_copy(erel_hbm.at[pl.ds(ebase, 16)], relbuf_v)
            pltpu.sync_copy(et_hbm.at[pl.ds(ebase, 16)], tbuf_v)
            pltpu.sync_copy(esrc_hbm.at[pl.ds(ebase, 16)], srcbuf_v)
            rel = relbuf_v[...]
            t = tbuf_v[...]
            temf = tem.astype(jnp.float32)
            sw = jnp.exp(-t)
            lw = jnp.exp(t * (-0.1))
            inbf = jnp.where(rel < 4, temf, 0.0)
            outbf = temf - inbf
            w_s = sw * temf
            w_l = lw * temf
            gidbuf_v[...] = gid
            gidbuf2_v[...] = gid + jnp.where(rel < 4, 0, 1024)
            lanes = lax.broadcasted_iota(jnp.int32, (16,), 0)
            qs = [temf, t * temf, inbf, outbf,
                  sw * inbf, sw * outbf, lw * inbf, lw * outbf]
            for q in range(8):
                qrow = jnp.full((16,), q, jnp.int32)
                for i in range(16):
                    plsc.addupdate_scatter(accs_v, [qrow, gid], qs[q],
                                           mask=lanes == i)
            pltpu.sync_copy(erepr_hbm.at[pl.ds(ebase, 16)], erows_v)
            pltpu.async_copy(nrepr_hbm.at[srcbuf_v], nrows_v, sem).wait()
            for i in range(16):
                for j in range(8):
                    er = erows_v[i, pl.ds(j * 16, 16)]
                    nr = nrows_v[i, pl.ds(j * 16, 16)]
                    eb0_v[i, pl.ds(j * 16, 16)] = w_s[i] * er
                    eb1_v[i, pl.ds(j * 16, 16)] = w_l[i] * er
                    nb0_v[i, pl.ds(j * 16, 16)] = w_s[i] * nr
                    nb1_v[i, pl.ds(j * 16, 16)] = w_l[i] * nr
            pltpu.sync_copy(eb0_v, acc_s.at[gidbuf2_v], add=True)
            pltpu.sync_copy(eb1_v, acc_l.at[gidbuf2_v], add=True)
            pltpu.sync_copy(nb0_v, acc_h1s.at[gidbuf_v], add=True)
            pltpu.sync_copy(nb1_v, acc_h1l.at[gidbuf_v], add=True)
        return 0

    def edge_block(b, _):
        base = b * EB * 16
        hit = None
        for v in range(EB):
            gid = egid_v[pl.ds(base + v * 16, 16)]
            dst = edst_v[pl.ds(base + v * 16, 16)]
            tgt = plsc.load_gather(tli_v, [gid])
            m = dst == tgt
            hit = m if hit is None else jnp.logical_or(hit, m)

        @pl.when(jnp.any(hit))
        def _():
            lax.fori_loop(b * EB, (b + 1) * EB, edge_slow, 0)
        return 0

    lax.fori_loop(0, EPW // 16 // EB, edge_block, 0)

    lanes = lax.broadcasted_iota(jnp.int32, (16,), 0)
    row8 = jnp.full((16,), 8, jnp.int32)
    row9 = jnp.full((16,), 9, jnp.int32)

    def _node_vreg(k, src_base):
        gid = ngid_v[pl.ds(k * 16, 16)]
        depth = ndepth_v[pl.ds(k * 16, 16)]
        hop1 = (depth == 1).astype(jnp.float32)
        hop2 = (depth >= 2).astype(jnp.float32)
        hop_any = hop1 + hop2
        idx = gid + jnp.where(depth >= 2, 1024, 0)
        for i in range(16):
            plsc.addupdate_scatter(accs_v, [row8, gid], hop1,
                                   mask=lanes == i)
            plsc.addupdate_scatter(accs_v, [row9, gid], hop2,
                                   mask=lanes == i)
        for i in range(16):
            for j in range(8):
                nr = zbuf_v[src_base + i, pl.ds(j * 16, 16)]
                zbuf_v[src_base + i, pl.ds(j * 16, 16)] = hop_any[i] * nr
        return idx

    ngroups = jnp.where(wid < 31, NODE_CHUNK // 64, 1)

    def node_group(gi, _):
        pltpu.sync_copy(nrepr_hbm.at[pl.ds(n0 + gi * 64, 64)],
                        zbuf_v.at[pl.ds(0, 64)])
        for v in range(4):
            idx = _node_vreg(gi * 4 + v, v * 16)
            ngidbuf64_v[pl.ds(v * 16, 16)] = idx
        pltpu.sync_copy(zbuf_v, acc_hop.at[ngidbuf64_v], add=True)
        return 0

    lax.fori_loop(0, ngroups, node_group, 0)

    @pl.when(wid == 31)
    def _():
        pltpu.sync_copy(nrepr_hbm.at[pl.ds(n0 + 64, 16)],
                        zbuf_v.at[pl.ds(0, 16)])
        idx = _node_vreg(4, 0)
        ngidbuf_v[...] = idx
        pltpu.sync_copy(zbuf_v.at[pl.ds(0, 16)],
                        acc_hop.at[ngidbuf_v], add=True)

    pltpu.sync_copy(accs_v, scal_out.at[wid])
    plsc.subcore_barrier()
    halves = [(acc_s, 0), (acc_s, 1024), (acc_l, 0), (acc_l, 1024),
              (acc_h1s, 0), (acc_h1l, 0), (acc_hop, 0), (acc_hop, 1024)]
    wcps = [pltpu.async_copy(a.at[pl.ds(off + s * 64, 64)],
                             pooled_out.at[c, slot, pl.ds(s * 64, 64)], sem)
            for slot, (a, off) in enumerate(halves)]
    for cp in wcps:
        cp.wait()


def _sc_stage(node_repr, edge_repr, edge_src, edge_dst, rel_ids,
              edge_relative_time, target_local_idx, node_subgraph_id,
              node_hop_depth, edge_subgraph_id):
    mesh = plsc.VectorSubcoreMesh(core_axis_name="c", subcore_axis_name="s")
    f32 = jnp.float32
    i32 = jnp.int32
    fn = pl.kernel(
        _sc_kernel,
        mesh=mesh,
        compiler_params=pltpu.CompilerParams(needs_layout_passes=False),
        out_type=[
            jax.ShapeDtypeStruct((2, 8, N_TARGETS, HIDDEN), f32),
            jax.ShapeDtypeStruct((32, NQ, N_TARGETS), f32),
        ],
        scratch_types=[
            pltpu.VMEM((EPW,), i32),
            pltpu.VMEM((EPW,), i32),
            pltpu.VMEM((N_TARGETS,), i32),
            pltpu.VMEM((NODE_CHUNK,), i32),
            pltpu.VMEM((NODE_CHUNK,), i32),
            pltpu.VMEM((16, HIDDEN), f32),
            pltpu.VMEM((16, HIDDEN), f32),
            pltpu.VMEM((16,), i32),
            pltpu.VMEM((16,), f32),
            pltpu.VMEM((16,), i32),
            pltpu.VMEM((16,), i32),
            pltpu.VMEM((16,), i32),
            pltpu.VMEM((16,), i32),
            pltpu.VMEM((64,), i32),
            pltpu.VMEM((16, HIDDEN), f32),
            pltpu.VMEM((16, HIDDEN), f32),
            pltpu.VMEM((16, HIDDEN), f32),
            pltpu.VMEM((16, HIDDEN), f32),
            pltpu.VMEM((NQ, N_TARGETS), f32),
            pltpu.VMEM((64, HIDDEN), f32),
            pltpu.VMEM_SHARED((2 * N_TARGETS, HIDDEN), f32),
            pltpu.VMEM_SHARED((2 * N_TARGETS, HIDDEN), f32),
            pltpu.VMEM_SHARED((N_TARGETS, HIDDEN), f32),
            pltpu.VMEM_SHARED((N_TARGETS, HIDDEN), f32),
            pltpu.VMEM_SHARED((2 * N_TARGETS, HIDDEN), f32),
            pltpu.SemaphoreType.DMA,
        ],
    )
    return fn(edge_subgraph_id, edge_dst, rel_ids, edge_relative_time,
              edge_src, target_local_idx, node_subgraph_id,
              node_hop_depth, edge_repr, node_repr)


def _layer_norm(x, eps=1e-5):
    mu = jnp.mean(x, axis=-1, keepdims=True)
    var = jnp.mean((x - mu) ** 2, axis=-1, keepdims=True)
    return (x - mu) * lax.rsqrt(var + eps)


def _gelu(x):
    return x * 0.5 * (1.0 + lax.erf(x * 0.7071067811865476))


def _tc_kernel(pooled_ref, scal_ref, wsc_ref, bsc_ref, w1_ref, b1_ref,
               w2_ref, b2_ref, out_ref):
    pooled = pooled_ref[0] + pooled_ref[1]
    scal = jnp.sum(scal_ref[...], axis=0)
    eps = 1e-6
    sw_in, sw_out = scal[4], scal[5]
    lw_in, lw_out = scal[6], scal[7]
    smass = sw_in + sw_out
    lmass = lw_in + lw_out

    def mean(p, d):
        return p / jnp.maximum(d, eps)[:, None]

    in_s = mean(pooled[0], sw_in)
    out_s = mean(pooled[1], sw_out)
    in_l = mean(pooled[2], lw_in)
    out_l = mean(pooled[3], lw_out)
    h1e_s = mean(pooled[4], smass)
    h1e_l = mean(pooled[5], lmass)
    hop1m = mean(pooled[6], scal[8])
    hop2m = mean(pooled[7], scal[9])

    dgap = _layer_norm(out_l - in_l)
    feats = [
        _layer_norm(in_s + out_s - (in_l + out_l)),
        dgap,
        _layer_norm(hop1m - hop2m),
        _layer_norm(h1e_s - h1e_l),
        _layer_norm(jnp.abs(dgap)),
        _layer_norm(hop1m),
        _layer_norm(hop2m),
        _layer_norm(in_l + out_l),
    ]
    sf = jnp.stack([
        jnp.log1p(scal[2]), jnp.log1p(scal[3]),
        jnp.log1p(scal[8]), jnp.log1p(scal[9]),
        scal[1] / jnp.maximum(scal[0], eps),
        smass, lmass, smass - lmass,
    ], axis=-1)
    emb = _gelu(jnp.dot(sf, wsc_ref[...],
                        preferred_element_type=jnp.float32) + bsc_ref[...])
    h = jnp.dot(emb, w1_ref[pl.ds(8 * HIDDEN, 32)],
                preferred_element_type=jnp.float32)
    for k in range(8):
        h = h + jnp.dot(feats[k], w1_ref[pl.ds(k * HIDDEN, HIDDEN)],
                        preferred_element_type=jnp.float32)
    h = _gelu(h + b1_ref[...])
    out_ref[...] = 0.25 * jnp.tanh(
        jnp.dot(h, w2_ref[...], preferred_element_type=jnp.float32)
        + b2_ref[...])


def _tc_stage(pooled, scal, W_scalar, b_scalar, W1, b1, W2, b2):
    B = 256
    grid = N_TARGETS // B
    return pl.pallas_call(
        _tc_kernel,
        grid=(grid,),
        in_specs=[
            pl.BlockSpec((2, 8, B, HIDDEN), lambda i: (0, 0, i, 0)),
            pl.BlockSpec((32, NQ, B), lambda i: (0, 0, i)),
            pl.BlockSpec((8, 32), lambda i: (0, 0)),
            pl.BlockSpec((1, 32), lambda i: (0, 0)),
            pl.BlockSpec((8 * HIDDEN + 32, 2 * HIDDEN), lambda i: (0, 0)),
            pl.BlockSpec((1, 2 * HIDDEN), lambda i: (0, 0)),
            pl.BlockSpec((2 * HIDDEN, HIDDEN), lambda i: (0, 0)),
            pl.BlockSpec((1, HIDDEN), lambda i: (0, 0)),
        ],
        out_specs=pl.BlockSpec((B, HIDDEN), lambda i: (i, 0)),
        out_shape=jax.ShapeDtypeStruct((N_TARGETS, HIDDEN), jnp.float32),
    )(pooled, scal, W_scalar, b_scalar.reshape(1, -1), W1,
      b1.reshape(1, -1), W2, b2.reshape(1, -1))


def kernel(node_repr, edge_repr, edge_src, edge_dst, rel_ids,
           edge_relative_time, target_local_idx, node_subgraph_id,
           edge_subgraph_id, node_hop_depth, W_scalar, b_scalar,
           W1, b1, W2, b2):
    i32 = jnp.int32
    pooled, scal = _sc_stage(
        node_repr, edge_repr, edge_src.astype(i32), edge_dst.astype(i32),
        rel_ids.astype(i32), edge_relative_time,
        target_local_idx.astype(i32), node_subgraph_id.astype(i32),
        node_hop_depth.astype(i32), edge_subgraph_id.astype(i32))
    return _tc_stage(pooled, scal, W_scalar, b_scalar, W1, b1, W2, b2)


# speedup vs baseline: 103.4670x; 1.0391x over previous
import jax
import jax.numpy as jnp
from jax import lax
from jax.experimental import pallas as pl
from jax.experimental.pallas import tpu as pltpu
from jax.experimental.pallas import tpu_sc as plsc

HIDDEN = 128
N_NODES = 10000
N_EDGES = 320000
N_TARGETS = 1024
NQ = 10
EPW = N_EDGES // 32
NODE_CHUNK = 320


def _sc_kernel(egid_hbm, edst_hbm, erel_hbm, et_hbm, esrc_hbm, tli_hbm,
               ngid_hbm, ndepth_hbm, erepr_hbm, nrepr_hbm,
               pooled_out, scal_out,
               egid_v, edst_v, tli_v, ngid_v, ndepth_v,
               erows_v, nrows_v, relbuf_v, tbuf_v, srcbuf_v, gidbuf_v,
               gidbuf2_v, ngidbuf_v, ngidbuf64_v,
               eb0_v, eb1_v, nb0_v, nb1_v,
               accs_v, zbuf_v,
               acc_s, acc_l, acc_h1s, acc_h1l, acc_hop,
               sem):
    c = lax.axis_index("c")
    s = lax.axis_index("s")
    wid = c * 16 + s
    zero16 = jnp.zeros((16,), jnp.float32)

    def _zero_zbuf(i, _):
        for j in range(8):
            zbuf_v[i, pl.ds(j * 16, 16)] = zero16
        return 0

    lax.fori_loop(0, 64, _zero_zbuf, 0)

    def _zero_accs(i, _):
        for q in range(NQ):
            accs_v[q, pl.ds(i * 16, 16)] = zero16
        return 0

    lax.fori_loop(0, N_TARGETS // 16, _zero_accs, 0)

    e0 = wid * EPW
    n0 = wid * NODE_CHUNK
    zcps = [
        pltpu.async_copy(zbuf_v, acc_s.at[pl.ds(s * 128, 64)], sem),
        pltpu.async_copy(zbuf_v, acc_s.at[pl.ds(s * 128 + 64, 64)], sem),
        pltpu.async_copy(zbuf_v, acc_l.at[pl.ds(s * 128, 64)], sem),
        pltpu.async_copy(zbuf_v, acc_l.at[pl.ds(s * 128 + 64, 64)], sem),
        pltpu.async_copy(zbuf_v, acc_hop.at[pl.ds(s * 128, 64)], sem),
        pltpu.async_copy(zbuf_v, acc_hop.at[pl.ds(s * 128 + 64, 64)], sem),
        pltpu.async_copy(zbuf_v, acc_h1s.at[pl.ds(s * 64, 64)], sem),
        pltpu.async_copy(zbuf_v, acc_h1l.at[pl.ds(s * 64, 64)], sem),
    ]
    for cp in zcps:
        cp.wait()

    @pl.when(s == 0)
    def _():
        pltpu.sync_copy(zbuf_v.at[pl.ds(0, 16)],
                        acc_hop.at[pl.ds(2 * N_TARGETS, 16)])

    plsc.subcore_barrier()

    scps = [pltpu.async_copy(egid_hbm.at[pl.ds(e0, EPW)], egid_v, sem),
            pltpu.async_copy(edst_hbm.at[pl.ds(e0, EPW)], edst_v, sem),
            pltpu.async_copy(tli_hbm, tli_v, sem)]
    for cp in scps:
        cp.wait()

    @pl.when(wid < 31)
    def _():
        pltpu.sync_copy(ngid_hbm.at[pl.ds(n0, NODE_CHUNK)], ngid_v)
        pltpu.sync_copy(ndepth_hbm.at[pl.ds(n0, NODE_CHUNK)], ndepth_v)

    @pl.when(wid == 31)
    def _():
        pltpu.sync_copy(ngid_hbm.at[pl.ds(n0, 80)], ngid_v.at[pl.ds(0, 80)])
        pltpu.sync_copy(ndepth_hbm.at[pl.ds(n0, 80)],
                        ndepth_v.at[pl.ds(0, 80)])

    EB = 25

    def edge_slow(k, _):
        gid = egid_v[pl.ds(k * 16, 16)]
        dst = edst_v[pl.ds(k * 16, 16)]
        tgt = plsc.load_gather(tli_v, [gid])
        tem = dst == tgt

        @pl.when(jnp.any(tem))
        def _():
            ebase = e0 + k * 16
            pltpu.sync_copy(erel_hbm.at[pl.ds(ebase, 16)], relbuf_v)
            pltpu.sync_copy(et_hbm.at[pl.ds(ebase, 16)], tbuf_v)
            pltpu.sync_copy(esrc_hbm.at[pl.ds(ebase, 16)], srcbuf_v)
            rel = relbuf_v[...]
            t = tbuf_v[...]
            temf = tem.astype(jnp.float32)
            sw = jnp.exp(-t)
            lw = jnp.exp(t * (-0.1))
            inbf = jnp.where(rel < 4, temf, 0.0)
            outbf = temf - inbf
            w_s = sw * temf
            w_l = lw * temf
            gidbuf_v[...] = gid
            gidbuf2_v[...] = gid + jnp.where(rel < 4, 0, 1024)
            lanes = lax.broadcasted_iota(jnp.int32, (16,), 0)
            qs = [temf, t * temf, inbf, outbf,
                  sw * inbf, sw * outbf, lw * inbf, lw * outbf]
            for q in range(8):
                qrow = jnp.full((16,), q, jnp.int32)
                for i in range(16):
                    plsc.addupdate_scatter(accs_v, [qrow, gid], qs[q],
                                           mask=lanes == i)
            pltpu.sync_copy(erepr_hbm.at[pl.ds(ebase, 16)], erows_v)
            pltpu.async_copy(nrepr_hbm.at[srcbuf_v], nrows_v, sem).wait()
            for i in range(16):
                for j in range(8):
                    er = erows_v[i, pl.ds(j * 16, 16)]
                    nr = nrows_v[i, pl.ds(j * 16, 16)]
                    eb0_v[i, pl.ds(j * 16, 16)] = w_s[i] * er
                    eb1_v[i, pl.ds(j * 16, 16)] = w_l[i] * er
                    nb0_v[i, pl.ds(j * 16, 16)] = w_s[i] * nr
                    nb1_v[i, pl.ds(j * 16, 16)] = w_l[i] * nr
            pltpu.sync_copy(eb0_v, acc_s.at[gidbuf2_v], add=True)
            pltpu.sync_copy(eb1_v, acc_l.at[gidbuf2_v], add=True)
            pltpu.sync_copy(nb0_v, acc_h1s.at[gidbuf_v], add=True)
            pltpu.sync_copy(nb1_v, acc_h1l.at[gidbuf_v], add=True)
        return 0

    def edge_block(b, _):
        base = b * EB * 16
        hit = None
        for v in range(EB):
            gid = egid_v[pl.ds(base + v * 16, 16)]
            dst = edst_v[pl.ds(base + v * 16, 16)]
            tgt = plsc.load_gather(tli_v, [gid])
            m = dst == tgt
            hit = m if hit is None else jnp.logical_or(hit, m)

        @pl.when(jnp.any(hit))
        def _():
            lax.fori_loop(b * EB, (b + 1) * EB, edge_slow, 0)
        return 0

    lax.fori_loop(0, EPW // 16 // EB, edge_block, 0)

    lanes = lax.broadcasted_iota(jnp.int32, (16,), 0)
    row8 = jnp.full((16,), 8, jnp.int32)
    row9 = jnp.full((16,), 9, jnp.int32)

    def _node_vreg(k):
        gid = ngid_v[pl.ds(k * 16, 16)]
        depth = ndepth_v[pl.ds(k * 16, 16)]
        hop1 = (depth == 1).astype(jnp.float32)
        hop2 = (depth >= 2).astype(jnp.float32)
        idx = jnp.where(depth == 0, 2048,
                        gid + jnp.where(depth >= 2, 1024, 0))
        for i in range(16):
            plsc.addupdate_scatter(accs_v, [row8, gid], hop1,
                                   mask=lanes == i)
            plsc.addupdate_scatter(accs_v, [row9, gid], hop2,
                                   mask=lanes == i)
        return idx

    ngroups = jnp.where(wid < 31, NODE_CHUNK // 64, 1)

    def node_group(gi, _):
        pltpu.sync_copy(nrepr_hbm.at[pl.ds(n0 + gi * 64, 64)],
                        zbuf_v.at[pl.ds(0, 64)])
        for v in range(4):
            idx = _node_vreg(gi * 4 + v)
            ngidbuf64_v[pl.ds(v * 16, 16)] = idx
        pltpu.sync_copy(zbuf_v, acc_hop.at[ngidbuf64_v], add=True)
        return 0

    lax.fori_loop(0, ngroups, node_group, 0)

    @pl.when(wid == 31)
    def _():
        pltpu.sync_copy(nrepr_hbm.at[pl.ds(n0 + 64, 16)],
                        zbuf_v.at[pl.ds(0, 16)])
        idx = _node_vreg(4)
        ngidbuf_v[...] = idx
        pltpu.sync_copy(zbuf_v.at[pl.ds(0, 16)],
                        acc_hop.at[ngidbuf_v], add=True)

    pltpu.sync_copy(accs_v, scal_out.at[wid])
    plsc.subcore_barrier()
    halves = [(acc_s, 0), (acc_s, 1024), (acc_l, 0), (acc_l, 1024),
              (acc_h1s, 0), (acc_h1l, 0), (acc_hop, 0), (acc_hop, 1024)]
    wcps = [pltpu.async_copy(a.at[pl.ds(off + s * 64, 64)],
                             pooled_out.at[c, slot, pl.ds(s * 64, 64)], sem)
            for slot, (a, off) in enumerate(halves)]
    for cp in wcps:
        cp.wait()


def _sc_stage(node_repr, edge_repr, edge_src, edge_dst, rel_ids,
              edge_relative_time, target_local_idx, node_subgraph_id,
              node_hop_depth, edge_subgraph_id):
    mesh = plsc.VectorSubcoreMesh(core_axis_name="c", subcore_axis_name="s")
    f32 = jnp.float32
    i32 = jnp.int32
    fn = pl.kernel(
        _sc_kernel,
        mesh=mesh,
        compiler_params=pltpu.CompilerParams(needs_layout_passes=False),
        out_type=[
            jax.ShapeDtypeStruct((2, 8, N_TARGETS, HIDDEN), f32),
            jax.ShapeDtypeStruct((32, NQ, N_TARGETS), f32),
        ],
        scratch_types=[
            pltpu.VMEM((EPW,), i32),
            pltpu.VMEM((EPW,), i32),
            pltpu.VMEM((N_TARGETS,), i32),
            pltpu.VMEM((NODE_CHUNK,), i32),
            pltpu.VMEM((NODE_CHUNK,), i32),
            pltpu.VMEM((16, HIDDEN), f32),
            pltpu.VMEM((16, HIDDEN), f32),
            pltpu.VMEM((16,), i32),
            pltpu.VMEM((16,), f32),
            pltpu.VMEM((16,), i32),
            pltpu.VMEM((16,), i32),
            pltpu.VMEM((16,), i32),
            pltpu.VMEM((16,), i32),
            pltpu.VMEM((64,), i32),
            pltpu.VMEM((16, HIDDEN), f32),
            pltpu.VMEM((16, HIDDEN), f32),
            pltpu.VMEM((16, HIDDEN), f32),
            pltpu.VMEM((16, HIDDEN), f32),
            pltpu.VMEM((NQ, N_TARGETS), f32),
            pltpu.VMEM((64, HIDDEN), f32),
            pltpu.VMEM_SHARED((2 * N_TARGETS, HIDDEN), f32),
            pltpu.VMEM_SHARED((2 * N_TARGETS, HIDDEN), f32),
            pltpu.VMEM_SHARED((N_TARGETS, HIDDEN), f32),
            pltpu.VMEM_SHARED((N_TARGETS, HIDDEN), f32),
            pltpu.VMEM_SHARED((2 * N_TARGETS + 16, HIDDEN), f32),
            pltpu.SemaphoreType.DMA,
        ],
    )
    return fn(edge_subgraph_id, edge_dst, rel_ids, edge_relative_time,
              edge_src, target_local_idx, node_subgraph_id,
              node_hop_depth, edge_repr, node_repr)


def _layer_norm(x, eps=1e-5):
    mu = jnp.mean(x, axis=-1, keepdims=True)
    var = jnp.mean((x - mu) ** 2, axis=-1, keepdims=True)
    return (x - mu) * lax.rsqrt(var + eps)


def _gelu(x):
    return x * 0.5 * (1.0 + lax.erf(x * 0.7071067811865476))


def _tc_kernel(pooled_ref, scal_ref, wsc_ref, bsc_ref, w1_ref, b1_ref,
               w2_ref, b2_ref, out_ref):
    pooled = pooled_ref[0] + pooled_ref[1]
    scal = jnp.sum(scal_ref[...], axis=0)
    eps = 1e-6
    sw_in, sw_out = scal[4], scal[5]
    lw_in, lw_out = scal[6], scal[7]
    smass = sw_in + sw_out
    lmass = lw_in + lw_out

    def mean(p, d):
        return p / jnp.maximum(d, eps)[:, None]

    in_s = mean(pooled[0], sw_in)
    out_s = mean(pooled[1], sw_out)
    in_l = mean(pooled[2], lw_in)
    out_l = mean(pooled[3], lw_out)
    h1e_s = mean(pooled[4], smass)
    h1e_l = mean(pooled[5], lmass)
    hop1m = mean(pooled[6], scal[8])
    hop2m = mean(pooled[7], scal[9])

    dgap = _layer_norm(out_l - in_l)
    feats = [
        _layer_norm(in_s + out_s - (in_l + out_l)),
        dgap,
        _layer_norm(hop1m - hop2m),
        _layer_norm(h1e_s - h1e_l),
        _layer_norm(jnp.abs(dgap)),
        _layer_norm(hop1m),
        _layer_norm(hop2m),
        _layer_norm(in_l + out_l),
    ]
    sf = jnp.stack([
        jnp.log1p(scal[2]), jnp.log1p(scal[3]),
        jnp.log1p(scal[8]), jnp.log1p(scal[9]),
        scal[1] / jnp.maximum(scal[0], eps),
        smass, lmass, smass - lmass,
    ], axis=-1)
    emb = _gelu(jnp.dot(sf, wsc_ref[...],
                        preferred_element_type=jnp.float32) + bsc_ref[...])
    h = jnp.dot(emb, w1_ref[pl.ds(8 * HIDDEN, 32)],
                preferred_element_type=jnp.float32)
    for k in range(8):
        h = h + jnp.dot(feats[k], w1_ref[pl.ds(k * HIDDEN, HIDDEN)],
                        preferred_element_type=jnp.float32)
    h = _gelu(h + b1_ref[...])
    out_ref[...] = 0.25 * jnp.tanh(
        jnp.dot(h, w2_ref[...], preferred_element_type=jnp.float32)
        + b2_ref[...])


def _tc_stage(pooled, scal, W_scalar, b_scalar, W1, b1, W2, b2):
    B = 256
    grid = N_TARGETS // B
    return pl.pallas_call(
        _tc_kernel,
        grid=(grid,),
        in_specs=[
            pl.BlockSpec((2, 8, B, HIDDEN), lambda i: (0, 0, i, 0)),
            pl.BlockSpec((32, NQ, B), lambda i: (0, 0, i)),
            pl.BlockSpec((8, 32), lambda i: (0, 0)),
            pl.BlockSpec((1, 32), lambda i: (0, 0)),
            pl.BlockSpec((8 * HIDDEN + 32, 2 * HIDDEN), lambda i: (0, 0)),
            pl.BlockSpec((1, 2 * HIDDEN), lambda i: (0, 0)),
            pl.BlockSpec((2 * HIDDEN, HIDDEN), lambda i: (0, 0)),
            pl.BlockSpec((1, HIDDEN), lambda i: (0, 0)),
        ],
        out_specs=pl.BlockSpec((B, HIDDEN), lambda i: (i, 0)),
        out_shape=jax.ShapeDtypeStruct((N_TARGETS, HIDDEN), jnp.float32),
    )(pooled, scal, W_scalar, b_scalar.reshape(1, -1), W1,
      b1.reshape(1, -1), W2, b2.reshape(1, -1))


def kernel(node_repr, edge_repr, edge_src, edge_dst, rel_ids,
           edge_relative_time, target_local_idx, node_subgraph_id,
           edge_subgraph_id, node_hop_depth, W_scalar, b_scalar,
           W1, b1, W2, b2):
    i32 = jnp.int32
    pooled, scal = _sc_stage(
        node_repr, edge_repr, edge_src.astype(i32), edge_dst.astype(i32),
        rel_ids.astype(i32), edge_relative_time,
        target_local_idx.astype(i32), node_subgraph_id.astype(i32),
        node_hop_depth.astype(i32), edge_subgraph_id.astype(i32))
    return _tc_stage(pooled, scal, W_scalar, b_scalar, W1, b1, W2, b2)


# speedup vs baseline: 106.2475x; 1.0269x over previous
import jax
import jax.numpy as jnp
from jax import lax
from jax.experimental import pallas as pl
from jax.experimental.pallas import tpu as pltpu
from jax.experimental.pallas import tpu_sc as plsc

HIDDEN = 128
N_NODES = 10000
N_EDGES = 320000
N_TARGETS = 1024
NQ = 10
EPW = N_EDGES // 32
NODE_CHUNK = 320


def _sc_kernel(egid_hbm, edst_hbm, erel_hbm, et_hbm, esrc_hbm, tli_hbm,
               ngid_hbm, ndepth_hbm, erepr_hbm, nrepr_hbm,
               pooled_out, scal_out,
               egid_v, edst_v, tli_v, ngid_v, ndepth_v,
               srcbuf_v, relbuf_v, tbuf_v,
               erows_v, nrows_v, gidbuf_v,
               gidbuf2_v, gidbufh_v, ngidbuf_v, ngidbuf64_v,
               eb0_v, eb1_v, nb0_v, nb1_v,
               accs_v, zbuf_v,
               acc_s, acc_l, acc_h1s, acc_h1l, acc_hop,
               sem):
    c = lax.axis_index("c")
    s = lax.axis_index("s")
    wid = c * 16 + s
    zero16 = jnp.zeros((16,), jnp.float32)

    def _zero_zbuf(i, _):
        for j in range(8):
            zbuf_v[i, pl.ds(j * 16, 16)] = zero16
        return 0

    lax.fori_loop(0, 64, _zero_zbuf, 0)

    def _zero_accs(i, _):
        for q in range(NQ):
            accs_v[q, pl.ds(i * 16, 16)] = zero16
        return 0

    lax.fori_loop(0, N_TARGETS // 16, _zero_accs, 0)

    e0 = wid * EPW
    n0 = wid * NODE_CHUNK
    zcps = [
        pltpu.async_copy(zbuf_v, acc_s.at[pl.ds(s * 128, 64)], sem),
        pltpu.async_copy(zbuf_v, acc_s.at[pl.ds(s * 128 + 64, 64)], sem),
        pltpu.async_copy(zbuf_v, acc_l.at[pl.ds(s * 128, 64)], sem),
        pltpu.async_copy(zbuf_v, acc_l.at[pl.ds(s * 128 + 64, 64)], sem),
        pltpu.async_copy(zbuf_v, acc_hop.at[pl.ds(s * 128, 64)], sem),
        pltpu.async_copy(zbuf_v, acc_hop.at[pl.ds(s * 128 + 64, 64)], sem),
        pltpu.async_copy(zbuf_v, acc_h1s.at[pl.ds(s * 64, 64)], sem),
        pltpu.async_copy(zbuf_v, acc_h1l.at[pl.ds(s * 64, 64)], sem),
    ]
    for cp in zcps:
        cp.wait()

    def _zero_bufs(i, _):
        for j in range(8):
            eb0_v[i, pl.ds(j * 16, 16)] = zero16
            eb1_v[i, pl.ds(j * 16, 16)] = zero16
            nb0_v[i, pl.ds(j * 16, 16)] = zero16
            nb1_v[i, pl.ds(j * 16, 16)] = zero16
        return 0

    lax.fori_loop(0, 16, _zero_bufs, 0)

    @pl.when(s == 0)
    def _():
        pltpu.sync_copy(zbuf_v.at[pl.ds(0, 16)],
                        acc_hop.at[pl.ds(2 * N_TARGETS, 16)])

    plsc.subcore_barrier()

    scps = [pltpu.async_copy(egid_hbm.at[pl.ds(e0, EPW)], egid_v, sem),
            pltpu.async_copy(edst_hbm.at[pl.ds(e0, EPW)], edst_v, sem),
            pltpu.async_copy(tli_hbm, tli_v, sem)]
    for cp in scps:
        cp.wait()

    @pl.when(wid < 31)
    def _():
        pltpu.sync_copy(ngid_hbm.at[pl.ds(n0, NODE_CHUNK)], ngid_v)
        pltpu.sync_copy(ndepth_hbm.at[pl.ds(n0, NODE_CHUNK)], ndepth_v)

    @pl.when(wid == 31)
    def _():
        pltpu.sync_copy(ngid_hbm.at[pl.ds(n0, 80)], ngid_v.at[pl.ds(0, 80)])
        pltpu.sync_copy(ndepth_hbm.at[pl.ds(n0, 80)],
                        ndepth_v.at[pl.ds(0, 80)])

    EB = 25

    def edge_slow(k, _):
        gid = egid_v[pl.ds(k * 16, 16)]
        dst = edst_v[pl.ds(k * 16, 16)]
        tgt = plsc.load_gather(tli_v, [gid])
        tem = dst == tgt

        @pl.when(jnp.any(tem))
        def _():
            ebase = e0 + k * 16
            cpe = pltpu.async_copy(erepr_hbm.at[pl.ds(ebase, 16)],
                                   erows_v, sem)
            cps = pltpu.async_copy(esrc_hbm.at[pl.ds(ebase, 16)],
                                   srcbuf_v, sem)
            cpr = pltpu.async_copy(erel_hbm.at[pl.ds(ebase, 16)],
                                   relbuf_v, sem)
            cpt = pltpu.async_copy(et_hbm.at[pl.ds(ebase, 16)],
                                   tbuf_v, sem)
            cpe.wait()
            cps.wait()
            cpr.wait()
            cpt.wait()
            cpn = pltpu.async_copy(nrepr_hbm.at[srcbuf_v], nrows_v, sem)
            rel = relbuf_v[...]
            t = tbuf_v[...]
            temf = tem.astype(jnp.float32)
            sw = jnp.exp(-t)
            lw = jnp.exp(t * (-0.1))
            inbf = jnp.where(rel < 4, temf, 0.0)
            outbf = temf - inbf
            w_s = sw * temf
            w_l = lw * temf
            gidbuf_v[...] = gid
            gidbuf2_v[...] = gid + jnp.where(rel < 4, 0, 1024)
            gidbufh_v[...] = gid
            lanes = lax.broadcasted_iota(jnp.int32, (16,), 0)
            qs = [temf, t * temf, inbf, outbf,
                  sw * inbf, sw * outbf, lw * inbf, lw * outbf]
            cpn.wait()
            for i in range(16):
                @pl.when(temf[i] > 0)
                def _(i=i):
                    for q in range(8):
                        plsc.addupdate_scatter(
                            accs_v, [jnp.full((16,), q, jnp.int32), gid],
                            qs[q], mask=lanes == i)
                    for j in range(8):
                        er = erows_v[i, pl.ds(j * 16, 16)]
                        nr = nrows_v[i, pl.ds(j * 16, 16)]
                        eb0_v[i, pl.ds(j * 16, 16)] = w_s[i] * er
                        eb1_v[i, pl.ds(j * 16, 16)] = w_l[i] * er
                        nb0_v[i, pl.ds(j * 16, 16)] = w_s[i] * nr
                        nb1_v[i, pl.ds(j * 16, 16)] = w_l[i] * nr
            pltpu.sync_copy(eb0_v, acc_s.at[gidbuf2_v], add=True)
            pltpu.sync_copy(eb1_v, acc_l.at[gidbuf2_v], add=True)
            pltpu.sync_copy(nb0_v, acc_h1s.at[gidbufh_v], add=True)
            pltpu.sync_copy(nb1_v, acc_h1l.at[gidbufh_v], add=True)
            for i in range(16):
                @pl.when(temf[i] > 0)
                def _(i=i):
                    for j in range(8):
                        eb0_v[i, pl.ds(j * 16, 16)] = zero16
                        eb1_v[i, pl.ds(j * 16, 16)] = zero16
                        nb0_v[i, pl.ds(j * 16, 16)] = zero16
                        nb1_v[i, pl.ds(j * 16, 16)] = zero16
        return 0

    def edge_block(b, _):
        base = b * EB * 16
        hit = None
        for v in range(EB):
            gid = egid_v[pl.ds(base + v * 16, 16)]
            dst = edst_v[pl.ds(base + v * 16, 16)]
            tgt = plsc.load_gather(tli_v, [gid])
            m = dst == tgt
            hit = m if hit is None else jnp.logical_or(hit, m)

        @pl.when(jnp.any(hit))
        def _():
            lax.fori_loop(b * EB, (b + 1) * EB, edge_slow, 0)
        return 0

    lax.fori_loop(0, EPW // 16 // EB, edge_block, 0)

    lanes = lax.broadcasted_iota(jnp.int32, (16,), 0)
    row8 = jnp.full((16,), 8, jnp.int32)
    row9 = jnp.full((16,), 9, jnp.int32)

    def _node_vreg(k):
        gid = ngid_v[pl.ds(k * 16, 16)]
        depth = ndepth_v[pl.ds(k * 16, 16)]
        hop1 = (depth == 1).astype(jnp.float32)
        hop2 = (depth >= 2).astype(jnp.float32)
        idx = jnp.where(depth == 0, 2048,
                        gid + jnp.where(depth >= 2, 1024, 0))
        for i in range(16):
            plsc.addupdate_scatter(accs_v, [row8, gid], hop1,
                                   mask=lanes == i)
            plsc.addupdate_scatter(accs_v, [row9, gid], hop2,
                                   mask=lanes == i)
        return idx

    ngroups = jnp.where(wid < 31, NODE_CHUNK // 64, 1)

    def node_group(gi, _):
        pltpu.sync_copy(nrepr_hbm.at[pl.ds(n0 + gi * 64, 64)],
                        zbuf_v.at[pl.ds(0, 64)])
        for v in range(4):
            idx = _node_vreg(gi * 4 + v)
            ngidbuf64_v[pl.ds(v * 16, 16)] = idx
        pltpu.sync_copy(zbuf_v, acc_hop.at[ngidbuf64_v], add=True)
        return 0

    lax.fori_loop(0, ngroups, node_group, 0)

    @pl.when(wid == 31)
    def _():
        pltpu.sync_copy(nrepr_hbm.at[pl.ds(n0 + 64, 16)],
                        zbuf_v.at[pl.ds(0, 16)])
        idx = _node_vreg(4)
        ngidbuf_v[...] = idx
        pltpu.sync_copy(zbuf_v.at[pl.ds(0, 16)],
                        acc_hop.at[ngidbuf_v], add=True)

    pltpu.sync_copy(accs_v, scal_out.at[wid])
    plsc.subcore_barrier()
    halves = [(acc_s, 0), (acc_s, 1024), (acc_l, 0), (acc_l, 1024),
              (acc_h1s, 0), (acc_h1l, 0), (acc_hop, 0), (acc_hop, 1024)]
    wcps = [pltpu.async_copy(a.at[pl.ds(off + s * 64, 64)],
                             pooled_out.at[c, slot, pl.ds(s * 64, 64)], sem)
            for slot, (a, off) in enumerate(halves)]
    for cp in wcps:
        cp.wait()


def _sc_stage(node_repr, edge_repr, edge_src, edge_dst, rel_ids,
              edge_relative_time, target_local_idx, node_subgraph_id,
              node_hop_depth, edge_subgraph_id):
    mesh = plsc.VectorSubcoreMesh(core_axis_name="c", subcore_axis_name="s")
    f32 = jnp.float32
    i32 = jnp.int32
    fn = pl.kernel(
        _sc_kernel,
        mesh=mesh,
        compiler_params=pltpu.CompilerParams(needs_layout_passes=False),
        out_type=[
            jax.ShapeDtypeStruct((2, 8, N_TARGETS, HIDDEN), f32),
            jax.ShapeDtypeStruct((32, NQ, N_TARGETS), f32),
        ],
        scratch_types=[
            pltpu.VMEM((EPW,), i32),
            pltpu.VMEM((EPW,), i32),
            pltpu.VMEM((N_TARGETS,), i32),
            pltpu.VMEM((NODE_CHUNK,), i32),
            pltpu.VMEM((NODE_CHUNK,), i32),
            pltpu.VMEM((16,), i32),
            pltpu.VMEM((16,), i32),
            pltpu.VMEM((16,), f32),
            pltpu.VMEM((16, HIDDEN), f32),
            pltpu.VMEM((16, HIDDEN), f32),
            pltpu.VMEM((16,), i32),
            pltpu.VMEM((16,), i32),
            pltpu.VMEM((16,), i32),
            pltpu.VMEM((16,), i32),
            pltpu.VMEM((64,), i32),
            pltpu.VMEM((16, HIDDEN), f32),
            pltpu.VMEM((16, HIDDEN), f32),
            pltpu.VMEM((16, HIDDEN), f32),
            pltpu.VMEM((16, HIDDEN), f32),
            pltpu.VMEM((NQ, N_TARGETS), f32),
            pltpu.VMEM((64, HIDDEN), f32),
            pltpu.VMEM_SHARED((2 * N_TARGETS, HIDDEN), f32),
            pltpu.VMEM_SHARED((2 * N_TARGETS, HIDDEN), f32),
            pltpu.VMEM_SHARED((N_TARGETS, HIDDEN), f32),
            pltpu.VMEM_SHARED((N_TARGETS, HIDDEN), f32),
            pltpu.VMEM_SHARED((2 * N_TARGETS + 16, HIDDEN), f32),
            pltpu.SemaphoreType.DMA,
        ],
    )
    return fn(edge_subgraph_id, edge_dst, rel_ids, edge_relative_time,
              edge_src, target_local_idx, node_subgraph_id,
              node_hop_depth, edge_repr, node_repr)


def _layer_norm(x, eps=1e-5):
    mu = jnp.mean(x, axis=-1, keepdims=True)
    var = jnp.mean((x - mu) ** 2, axis=-1, keepdims=True)
    return (x - mu) * lax.rsqrt(var + eps)


def _gelu(x):
    return x * 0.5 * (1.0 + lax.erf(x * 0.7071067811865476))


def _tc_kernel(pooled_ref, scal_ref, wsc_ref, bsc_ref, w1_ref, b1_ref,
               w2_ref, b2_ref, out_ref):
    pooled = pooled_ref[0] + pooled_ref[1]
    scal = jnp.sum(scal_ref[...], axis=0)
    eps = 1e-6
    sw_in, sw_out = scal[4], scal[5]
    lw_in, lw_out = scal[6], scal[7]
    smass = sw_in + sw_out
    lmass = lw_in + lw_out

    def mean(p, d):
        return p / jnp.maximum(d, eps)[:, None]

    in_s = mean(pooled[0], sw_in)
    out_s = mean(pooled[1], sw_out)
    in_l = mean(pooled[2], lw_in)
    out_l = mean(pooled[3], lw_out)
    h1e_s = mean(pooled[4], smass)
    h1e_l = mean(pooled[5], lmass)
    hop1m = mean(pooled[6], scal[8])
    hop2m = mean(pooled[7], scal[9])

    dgap = _layer_norm(out_l - in_l)
    feats = [
        _layer_norm(in_s + out_s - (in_l + out_l)),
        dgap,
        _layer_norm(hop1m - hop2m),
        _layer_norm(h1e_s - h1e_l),
        _layer_norm(jnp.abs(dgap)),
        _layer_norm(hop1m),
        _layer_norm(hop2m),
        _layer_norm(in_l + out_l),
    ]
    sf = jnp.stack([
        jnp.log1p(scal[2]), jnp.log1p(scal[3]),
        jnp.log1p(scal[8]), jnp.log1p(scal[9]),
        scal[1] / jnp.maximum(scal[0], eps),
        smass, lmass, smass - lmass,
    ], axis=-1)
    emb = _gelu(jnp.dot(sf, wsc_ref[...],
                        preferred_element_type=jnp.float32) + bsc_ref[...])
    h = jnp.dot(emb, w1_ref[pl.ds(8 * HIDDEN, 32)],
                preferred_element_type=jnp.float32)
    for k in range(8):
        h = h + jnp.dot(feats[k], w1_ref[pl.ds(k * HIDDEN, HIDDEN)],
                        preferred_element_type=jnp.float32)
    h = _gelu(h + b1_ref[...])
    out_ref[...] = 0.25 * jnp.tanh(
        jnp.dot(h, w2_ref[...], preferred_element_type=jnp.float32)
        + b2_ref[...])


def _tc_stage(pooled, scal, W_scalar, b_scalar, W1, b1, W2, b2):
    B = 256
    grid = N_TARGETS // B
    return pl.pallas_call(
        _tc_kernel,
        grid=(grid,),
        in_specs=[
            pl.BlockSpec((2, 8, B, HIDDEN), lambda i: (0, 0, i, 0)),
            pl.BlockSpec((32, NQ, B), lambda i: (0, 0, i)),
            pl.BlockSpec((8, 32), lambda i: (0, 0)),
            pl.BlockSpec((1, 32), lambda i: (0, 0)),
            pl.BlockSpec((8 * HIDDEN + 32, 2 * HIDDEN), lambda i: (0, 0)),
            pl.BlockSpec((1, 2 * HIDDEN), lambda i: (0, 0)),
            pl.BlockSpec((2 * HIDDEN, HIDDEN), lambda i: (0, 0)),
            pl.BlockSpec((1, HIDDEN), lambda i: (0, 0)),
        ],
        out_specs=pl.BlockSpec((B, HIDDEN), lambda i: (i, 0)),
        out_shape=jax.ShapeDtypeStruct((N_TARGETS, HIDDEN), jnp.float32),
    )(pooled, scal, W_scalar, b_scalar.reshape(1, -1), W1,
      b1.reshape(1, -1), W2, b2.reshape(1, -1))


def kernel(node_repr, edge_repr, edge_src, edge_dst, rel_ids,
           edge_relative_time, target_local_idx, node_subgraph_id,
           edge_subgraph_id, node_hop_depth, W_scalar, b_scalar,
           W1, b1, W2, b2):
    i32 = jnp.int32
    pooled, scal = _sc_stage(
        node_repr, edge_repr, edge_src.astype(i32), edge_dst.astype(i32),
        rel_ids.astype(i32), edge_relative_time,
        target_local_idx.astype(i32), node_subgraph_id.astype(i32),
        node_hop_depth.astype(i32), edge_subgraph_id.astype(i32))
    return _tc_stage(pooled, scal, W_scalar, b_scalar, W1, b1, W2, b2)


# speedup vs baseline: 108.6146x; 1.0223x over previous
import jax
import jax.numpy as jnp
from jax import lax
from jax.experimental import pallas as pl
from jax.experimental.pallas import tpu as pltpu
from jax.experimental.pallas import tpu_sc as plsc

HIDDEN = 128
N_NODES = 10000
N_EDGES = 320000
N_TARGETS = 1024
NQ = 10
EPW = N_EDGES // 32
NODE_CHUNK = 320


def _sc_kernel(egid_hbm, edst_hbm, erel_hbm, et_hbm, esrc_hbm, tli_hbm,
               ngid_hbm, ndepth_hbm, erepr_hbm, nrepr_hbm,
               pooled_out, scal_out,
               egid_v, edst_v, tli_v, ngid_v, ndepth_v,
               srcbuf_v, relbuf_v, tbuf_v,
               erows_v, nrows_v, gidbuf_v,
               gidbuf2_v, gidbufh_v, ngidbuf_v, ngidbuf64_v,
               eb0_v, eb1_v, nb0_v, nb1_v,
               accs_v, zbuf_v,
               acc_s, acc_l, acc_h1s, acc_h1l, acc_hop,
               sem):
    c = lax.axis_index("c")
    s = lax.axis_index("s")
    wid = c * 16 + s
    zero16 = jnp.zeros((16,), jnp.float32)

    def _zero_zbuf(i, _):
        for j in range(8):
            zbuf_v[i, pl.ds(j * 16, 16)] = zero16
        return 0

    lax.fori_loop(0, 64, _zero_zbuf, 0)

    def _zero_accs(i, _):
        for q in range(NQ):
            accs_v[q, pl.ds(i * 16, 16)] = zero16
        return 0

    lax.fori_loop(0, N_TARGETS // 16, _zero_accs, 0)

    e0 = wid * EPW
    n0 = wid * NODE_CHUNK
    zcps = [
        pltpu.async_copy(zbuf_v, acc_s.at[pl.ds(s * 128, 64)], sem),
        pltpu.async_copy(zbuf_v, acc_s.at[pl.ds(s * 128 + 64, 64)], sem),
        pltpu.async_copy(zbuf_v, acc_l.at[pl.ds(s * 128, 64)], sem),
        pltpu.async_copy(zbuf_v, acc_l.at[pl.ds(s * 128 + 64, 64)], sem),
        pltpu.async_copy(zbuf_v, acc_hop.at[pl.ds(s * 128, 64)], sem),
        pltpu.async_copy(zbuf_v, acc_hop.at[pl.ds(s * 128 + 64, 64)], sem),
        pltpu.async_copy(zbuf_v, acc_h1s.at[pl.ds(s * 64, 64)], sem),
        pltpu.async_copy(zbuf_v, acc_h1l.at[pl.ds(s * 64, 64)], sem),
    ]
    for cp in zcps:
        cp.wait()

    def _zero_bufs(i, _):
        for j in range(8):
            eb0_v[i, pl.ds(j * 16, 16)] = zero16
            eb1_v[i, pl.ds(j * 16, 16)] = zero16
            nb0_v[i, pl.ds(j * 16, 16)] = zero16
            nb1_v[i, pl.ds(j * 16, 16)] = zero16
        return 0

    lax.fori_loop(0, 16, _zero_bufs, 0)

    @pl.when(s == 0)
    def _():
        pltpu.sync_copy(zbuf_v.at[pl.ds(0, 16)],
                        acc_hop.at[pl.ds(2 * N_TARGETS, 16)])

    plsc.subcore_barrier()

    scps = [pltpu.async_copy(egid_hbm.at[pl.ds(e0, EPW)], egid_v, sem),
            pltpu.async_copy(edst_hbm.at[pl.ds(e0, EPW)], edst_v, sem),
            pltpu.async_copy(tli_hbm, tli_v, sem)]
    for cp in scps:
        cp.wait()

    @pl.when(wid < 31)
    def _():
        pltpu.sync_copy(ngid_hbm.at[pl.ds(n0, NODE_CHUNK)], ngid_v)
        pltpu.sync_copy(ndepth_hbm.at[pl.ds(n0, NODE_CHUNK)], ndepth_v)

    @pl.when(wid == 31)
    def _():
        pltpu.sync_copy(ngid_hbm.at[pl.ds(n0, 80)], ngid_v.at[pl.ds(0, 80)])
        pltpu.sync_copy(ndepth_hbm.at[pl.ds(n0, 80)],
                        ndepth_v.at[pl.ds(0, 80)])

    EB = 25

    def edge_slow(k, _):
        gid = egid_v[pl.ds(k * 16, 16)]
        dst = edst_v[pl.ds(k * 16, 16)]
        tgt = plsc.load_gather(tli_v, [gid])
        tem = dst == tgt

        @pl.when(jnp.any(tem))
        def _():
            ebase = e0 + k * 16
            cpe = pltpu.async_copy(erepr_hbm.at[pl.ds(ebase, 16)],
                                   erows_v, sem)
            cps = pltpu.async_copy(esrc_hbm.at[pl.ds(ebase, 16)],
                                   srcbuf_v, sem)
            cpr = pltpu.async_copy(erel_hbm.at[pl.ds(ebase, 16)],
                                   relbuf_v, sem)
            cpt = pltpu.async_copy(et_hbm.at[pl.ds(ebase, 16)],
                                   tbuf_v, sem)
            cpe.wait()
            cps.wait()
            cpr.wait()
            cpt.wait()
            cpn = pltpu.async_copy(nrepr_hbm.at[srcbuf_v], nrows_v, sem)
            rel = relbuf_v[...]
            t = tbuf_v[...]
            temf = tem.astype(jnp.float32)
            sw = jnp.exp(-t)
            lw = jnp.exp(t * (-0.1))
            inbf = jnp.where(rel < 4, temf, 0.0)
            outbf = temf - inbf
            w_s = sw * temf
            w_l = lw * temf
            gidbuf_v[...] = gid
            gidbuf2_v[...] = gid + jnp.where(rel < 4, 0, 1024)
            gidbufh_v[...] = gid
            lanes = lax.broadcasted_iota(jnp.int32, (16,), 0)
            qs = [temf, t * temf, inbf, outbf,
                  sw * inbf, sw * outbf, lw * inbf, lw * outbf]
            cpn.wait()
            for i in range(16):
                @pl.when(temf[i] > 0)
                def _(i=i):
                    for q in range(8):
                        plsc.addupdate_scatter(
                            accs_v, [jnp.full((16,), q, jnp.int32), gid],
                            qs[q], mask=lanes == i)
                    for j in range(8):
                        er = erows_v[i, pl.ds(j * 16, 16)]
                        nr = nrows_v[i, pl.ds(j * 16, 16)]
                        eb0_v[i, pl.ds(j * 16, 16)] = w_s[i] * er
                        eb1_v[i, pl.ds(j * 16, 16)] = w_l[i] * er
                        nb0_v[i, pl.ds(j * 16, 16)] = w_s[i] * nr
                        nb1_v[i, pl.ds(j * 16, 16)] = w_l[i] * nr
            acps = [
                pltpu.async_copy(eb0_v, acc_s.at[gidbuf2_v], sem, add=True),
                pltpu.async_copy(eb1_v, acc_l.at[gidbuf2_v], sem, add=True),
                pltpu.async_copy(nb0_v, acc_h1s.at[gidbufh_v], sem,
                                 add=True),
                pltpu.async_copy(nb1_v, acc_h1l.at[gidbufh_v], sem,
                                 add=True),
            ]
            for cp in acps:
                cp.wait()
            for i in range(16):
                @pl.when(temf[i] > 0)
                def _(i=i):
                    for j in range(8):
                        eb0_v[i, pl.ds(j * 16, 16)] = zero16
                        eb1_v[i, pl.ds(j * 16, 16)] = zero16
                        nb0_v[i, pl.ds(j * 16, 16)] = zero16
                        nb1_v[i, pl.ds(j * 16, 16)] = zero16
        return 0

    def edge_block(b, _):
        base = b * EB * 16
        hit = None
        for v in range(EB):
            gid = egid_v[pl.ds(base + v * 16, 16)]
            dst = edst_v[pl.ds(base + v * 16, 16)]
            tgt = plsc.load_gather(tli_v, [gid])
            m = dst == tgt
            hit = m if hit is None else jnp.logical_or(hit, m)

        @pl.when(jnp.any(hit))
        def _():
            lax.fori_loop(b * EB, (b + 1) * EB, edge_slow, 0)
        return 0

    lax.fori_loop(0, EPW // 16 // EB, edge_block, 0)

    lanes = lax.broadcasted_iota(jnp.int32, (16,), 0)
    row8 = jnp.full((16,), 8, jnp.int32)
    row9 = jnp.full((16,), 9, jnp.int32)

    def _node_vreg(k):
        gid = ngid_v[pl.ds(k * 16, 16)]
        depth = ndepth_v[pl.ds(k * 16, 16)]
        hop1 = (depth == 1).astype(jnp.float32)
        hop2 = (depth >= 2).astype(jnp.float32)
        idx = jnp.where(depth == 0, 2048,
                        gid + jnp.where(depth >= 2, 1024, 0))
        for i in range(16):
            plsc.addupdate_scatter(accs_v, [row8, gid], hop1,
                                   mask=lanes == i)
            plsc.addupdate_scatter(accs_v, [row9, gid], hop2,
                                   mask=lanes == i)
        return idx

    ngroups = jnp.where(wid < 31, NODE_CHUNK // 64, 1)

    def node_group(gi, _):
        pltpu.sync_copy(nrepr_hbm.at[pl.ds(n0 + gi * 64, 64)],
                        zbuf_v.at[pl.ds(0, 64)])
        for v in range(4):
            idx = _node_vreg(gi * 4 + v)
            ngidbuf64_v[pl.ds(v * 16, 16)] = idx
        pltpu.sync_copy(zbuf_v, acc_hop.at[ngidbuf64_v], add=True)
        return 0

    lax.fori_loop(0, ngroups, node_group, 0)

    @pl.when(wid == 31)
    def _():
        pltpu.sync_copy(nrepr_hbm.at[pl.ds(n0 + 64, 16)],
                        zbuf_v.at[pl.ds(0, 16)])
        idx = _node_vreg(4)
        ngidbuf_v[...] = idx
        pltpu.sync_copy(zbuf_v.at[pl.ds(0, 16)],
                        acc_hop.at[ngidbuf_v], add=True)

    pltpu.sync_copy(accs_v, scal_out.at[wid])
    plsc.subcore_barrier()
    halves = [(acc_s, 0), (acc_s, 1024), (acc_l, 0), (acc_l, 1024),
              (acc_h1s, 0), (acc_h1l, 0), (acc_hop, 0), (acc_hop, 1024)]
    wcps = [pltpu.async_copy(a.at[pl.ds(off + s * 64, 64)],
                             pooled_out.at[c, slot, pl.ds(s * 64, 64)], sem)
            for slot, (a, off) in enumerate(halves)]
    for cp in wcps:
        cp.wait()


def _sc_stage(node_repr, edge_repr, edge_src, edge_dst, rel_ids,
              edge_relative_time, target_local_idx, node_subgraph_id,
              node_hop_depth, edge_subgraph_id):
    mesh = plsc.VectorSubcoreMesh(core_axis_name="c", subcore_axis_name="s")
    f32 = jnp.float32
    i32 = jnp.int32
    fn = pl.kernel(
        _sc_kernel,
        mesh=mesh,
        compiler_params=pltpu.CompilerParams(needs_layout_passes=False),
        out_type=[
            jax.ShapeDtypeStruct((2, 8, N_TARGETS, HIDDEN), f32),
            jax.ShapeDtypeStruct((32, NQ, N_TARGETS), f32),
        ],
        scratch_types=[
            pltpu.VMEM((EPW,), i32),
            pltpu.VMEM((EPW,), i32),
            pltpu.VMEM((N_TARGETS,), i32),
            pltpu.VMEM((NODE_CHUNK,), i32),
            pltpu.VMEM((NODE_CHUNK,), i32),
            pltpu.VMEM((16,), i32),
            pltpu.VMEM((16,), i32),
            pltpu.VMEM((16,), f32),
            pltpu.VMEM((16, HIDDEN), f32),
            pltpu.VMEM((16, HIDDEN), f32),
            pltpu.VMEM((16,), i32),
            pltpu.VMEM((16,), i32),
            pltpu.VMEM((16,), i32),
            pltpu.VMEM((16,), i32),
            pltpu.VMEM((64,), i32),
            pltpu.VMEM((16, HIDDEN), f32),
            pltpu.VMEM((16, HIDDEN), f32),
            pltpu.VMEM((16, HIDDEN), f32),
            pltpu.VMEM((16, HIDDEN), f32),
            pltpu.VMEM((NQ, N_TARGETS), f32),
            pltpu.VMEM((64, HIDDEN), f32),
            pltpu.VMEM_SHARED((2 * N_TARGETS, HIDDEN), f32),
            pltpu.VMEM_SHARED((2 * N_TARGETS, HIDDEN), f32),
            pltpu.VMEM_SHARED((N_TARGETS, HIDDEN), f32),
            pltpu.VMEM_SHARED((N_TARGETS, HIDDEN), f32),
            pltpu.VMEM_SHARED((2 * N_TARGETS + 16, HIDDEN), f32),
            pltpu.SemaphoreType.DMA,
        ],
    )
    return fn(edge_subgraph_id, edge_dst, rel_ids, edge_relative_time,
              edge_src, target_local_idx, node_subgraph_id,
              node_hop_depth, edge_repr, node_repr)


def _layer_norm(x, eps=1e-5):
    mu = jnp.mean(x, axis=-1, keepdims=True)
    var = jnp.mean((x - mu) ** 2, axis=-1, keepdims=True)
    return (x - mu) * lax.rsqrt(var + eps)


def _gelu(x):
    return x * 0.5 * (1.0 + lax.erf(x * 0.7071067811865476))


def _tc_kernel(pooled_ref, scal_ref, wsc_ref, bsc_ref, w1_ref, b1_ref,
               w2_ref, b2_ref, out_ref):
    pooled = pooled_ref[0] + pooled_ref[1]
    scal = jnp.sum(scal_ref[...], axis=0)
    eps = 1e-6
    sw_in, sw_out = scal[4], scal[5]
    lw_in, lw_out = scal[6], scal[7]
    smass = sw_in + sw_out
    lmass = lw_in + lw_out

    def mean(p, d):
        return p / jnp.maximum(d, eps)[:, None]

    in_s = mean(pooled[0], sw_in)
    out_s = mean(pooled[1], sw_out)
    in_l = mean(pooled[2], lw_in)
    out_l = mean(pooled[3], lw_out)
    h1e_s = mean(pooled[4], smass)
    h1e_l = mean(pooled[5], lmass)
    hop1m = mean(pooled[6], scal[8])
    hop2m = mean(pooled[7], scal[9])

    dgap = _layer_norm(out_l - in_l)
    feats = [
        _layer_norm(in_s + out_s - (in_l + out_l)),
        dgap,
        _layer_norm(hop1m - hop2m),
        _layer_norm(h1e_s - h1e_l),
        _layer_norm(jnp.abs(dgap)),
        _layer_norm(hop1m),
        _layer_norm(hop2m),
        _layer_norm(in_l + out_l),
    ]
    sf = jnp.stack([
        jnp.log1p(scal[2]), jnp.log1p(scal[3]),
        jnp.log1p(scal[8]), jnp.log1p(scal[9]),
        scal[1] / jnp.maximum(scal[0], eps),
        smass, lmass, smass - lmass,
    ], axis=-1)
    emb = _gelu(jnp.dot(sf, wsc_ref[...],
                        preferred_element_type=jnp.float32) + bsc_ref[...])
    h = jnp.dot(emb, w1_ref[pl.ds(8 * HIDDEN, 32)],
                preferred_element_type=jnp.float32)
    for k in range(8):
        h = h + jnp.dot(feats[k], w1_ref[pl.ds(k * HIDDEN, HIDDEN)],
                        preferred_element_type=jnp.float32)
    h = _gelu(h + b1_ref[...])
    out_ref[...] = 0.25 * jnp.tanh(
        jnp.dot(h, w2_ref[...], preferred_element_type=jnp.float32)
        + b2_ref[...])


def _tc_stage(pooled, scal, W_scalar, b_scalar, W1, b1, W2, b2):
    B = 512
    grid = N_TARGETS // B
    return pl.pallas_call(
        _tc_kernel,
        grid=(grid,),
        in_specs=[
            pl.BlockSpec((2, 8, B, HIDDEN), lambda i: (0, 0, i, 0)),
            pl.BlockSpec((32, NQ, B), lambda i: (0, 0, i)),
            pl.BlockSpec((8, 32), lambda i: (0, 0)),
            pl.BlockSpec((1, 32), lambda i: (0, 0)),
            pl.BlockSpec((8 * HIDDEN + 32, 2 * HIDDEN), lambda i: (0, 0)),
            pl.BlockSpec((1, 2 * HIDDEN), lambda i: (0, 0)),
            pl.BlockSpec((2 * HIDDEN, HIDDEN), lambda i: (0, 0)),
            pl.BlockSpec((1, HIDDEN), lambda i: (0, 0)),
        ],
        out_specs=pl.BlockSpec((B, HIDDEN), lambda i: (i, 0)),
        out_shape=jax.ShapeDtypeStruct((N_TARGETS, HIDDEN), jnp.float32),
    )(pooled, scal, W_scalar, b_scalar.reshape(1, -1), W1,
      b1.reshape(1, -1), W2, b2.reshape(1, -1))


def kernel(node_repr, edge_repr, edge_src, edge_dst, rel_ids,
           edge_relative_time, target_local_idx, node_subgraph_id,
           edge_subgraph_id, node_hop_depth, W_scalar, b_scalar,
           W1, b1, W2, b2):
    i32 = jnp.int32
    pooled, scal = _sc_stage(
        node_repr, edge_repr, edge_src.astype(i32), edge_dst.astype(i32),
        rel_ids.astype(i32), edge_relative_time,
        target_local_idx.astype(i32), node_subgraph_id.astype(i32),
        node_hop_depth.astype(i32), edge_subgraph_id.astype(i32))
    return _tc_stage(pooled, scal, W_scalar, b_scalar, W1, b1, W2, b2)
